# baseline (speedup 1.0000x reference)
import functools

import jax
import jax.numpy as jnp
from jax import lax
from jax.experimental import pallas as pl
from jax.experimental.pallas import tpu as pltpu
from jax.experimental.pallas import tpu_sc as plsc

MOMENTUM = 0.8
NC, NS, L = 2, 16, 16
NW = NC * NS
C = 64
NBUF = 3
SELU = 4


@functools.cache
def _build(M, D, B):
    OWN = ((M + NW - 1) // NW + 7) // 8 * 8
    NFULL = (M - 1) // OWN
    TAILN = M - NFULL * OWN
    SEL = B + C + L
    mesh = plsc.VectorSubcoreMesh(
        core_axis_name="c", subcore_axis_name="s",
        num_cores=NC, num_subcores=NS)

    row_bufs = []
    for _ in range(NBUF):
        row_bufs += [
            pltpu.VMEM((C,), jnp.int32),
            pltpu.VMEM((C,), jnp.int32),
            pltpu.VMEM((C, D), jnp.float32),
            pltpu.VMEM((C, D), jnp.float32),
            pltpu.SemaphoreType.DMA,
            pltpu.SemaphoreType.DMA,
            pltpu.SemaphoreType.DMA,
        ]

    @functools.partial(
        pl.kernel, mesh=mesh,
        out_type=jax.ShapeDtypeStruct((M, D), jnp.float32),
        compiler_params=pltpu.CompilerParams(needs_layout_passes=False),
        scratch_types=[
            pltpu.VMEM((B,), jnp.int32),
            pltpu.VMEM((SEL,), jnp.int32),
            pltpu.VMEM((SEL,), jnp.int32),
            pltpu.VMEM((OWN + L,), jnp.int32),
            pltpu.SemaphoreType.DMA,
        ] + row_bufs,
    )
    def sc_update(mem_hbm, idx_hbm, val_hbm, out_hbm,
                  idx_v, sel_idx, sel_pos, aux, sem_c, *bufflat):
        bufs = [tuple(bufflat[i * 7:(i + 1) * 7]) for i in range(NBUF)]
        wid = lax.axis_index("s") * NC + lax.axis_index("c")
        lo = pl.multiple_of(wid * OWN, 8)
        hi = jnp.minimum(lo + OWN, M)
        lanes = lax.iota(jnp.int32, L)

        @pl.when(wid < NFULL)
        def _copy_full():
            pltpu.async_copy(mem_hbm.at[pl.ds(lo, OWN)],
                             out_hbm.at[pl.ds(lo, OWN)], sem_c)

        @pl.when(wid == NFULL)
        def _copy_tail():
            tlo = pl.multiple_of(NFULL * OWN, 8)
            pltpu.async_copy(mem_hbm.at[pl.ds(tlo, TAILN)],
                             out_hbm.at[pl.ds(tlo, TAILN)], sem_c)

        pltpu.sync_copy(idx_hbm, idx_v)

        def sel_body(i, off):
            for u in range(SELU):
                j = i * SELU + u
                v = idx_v[pl.ds(j * L, L)]
                msk = (v >= lo) & (v < hi)
                plsc.store_compressed(sel_idx.at[pl.ds(off, L)], v, mask=msk)
                plsc.store_compressed(sel_pos.at[pl.ds(off, L)],
                                      lanes + j * L, mask=msk)
                off = off + jnp.sum(msk.astype(jnp.int32))
            return off

        n = lax.fori_loop(0, B // (L * SELU), sel_body, jnp.int32(0))
        nvec = pl.cdiv(n, L)

        def _local(i):
            raw = sel_idx[pl.ds(i * L, L)]
            vl = jnp.minimum(jnp.maximum(raw - lo, 0), OWN - 1)
            vp = sel_pos[pl.ds(i * L, L)]
            valid = (lanes + i * L) < n
            return raw, vl, vp, valid

        def scat0(i, c_):
            _, vl, vp, valid = _local(i)
            plsc.store_scatter(aux, [vl], vp, mask=valid)
            return c_

        lax.fori_loop(0, nvec, scat0, jnp.int32(0))

        def fix_cond(state):
            return state[0] > 0

        def fix_body(state):
            _, it = state

            def chk(i, pend):
                _, vl, vp, valid = _local(i)
                cur = plsc.load_gather(aux, [vl])
                need = valid & (cur < vp)
                plsc.store_scatter(aux, [vl], vp, mask=need)
                return pend + jnp.sum(need.astype(jnp.int32))

            pend = lax.fori_loop(0, nvec, chk, jnp.int32(0))
            return pend, it + 1

        lax.while_loop(fix_cond, fix_body, (jnp.int32(1), jnp.int32(0)))

        def win_body(i, m):
            raw, vl, vp, valid = _local(i)
            cur = plsc.load_gather(aux, [vl])
            win = valid & (cur == vp)
            plsc.store_compressed(sel_idx.at[pl.ds(m, L)], raw, mask=win)
            plsc.store_compressed(sel_pos.at[pl.ds(m, L)], vp, mask=win)
            return m + jnp.sum(win.astype(jnp.int32))

        nwin = lax.fori_loop(0, nvec, win_body, jnp.int32(0))

        def stage(c_, buf):
            ci, cp, g, v, sg, sv, ss = buf
            for k in range(C // L):
                ci[pl.ds(k * L, L)] = sel_idx[pl.ds(c_ * C + k * L, L)]
                cp[pl.ds(k * L, L)] = sel_pos[pl.ds(c_ * C + k * L, L)]
            pltpu.async_copy(mem_hbm.at[ci], g, sg)
            pltpu.async_copy(val_hbm.at[cp], v, sv)

        def wait_scatter(buf):
            ci, cp, g, v, sg, sv, ss = buf
            pltpu.make_async_copy(g, out_hbm.at[ci], ss).wait()

        def blend_scatter(buf):
            ci, cp, g, v, sg, sv, ss = buf
            pltpu.make_async_copy(mem_hbm.at[ci], g, sg).wait()
            pltpu.make_async_copy(val_hbm.at[cp], v, sv).wait()

            def brow(r, carry):
                for d in range(D // L):
                    sl = pl.ds(d * L, L)
                    g[r, sl] = ((1.0 - MOMENTUM) * g[r, sl]
                                + MOMENTUM * v[r, sl])
                return carry

            lax.fori_loop(0, C, brow, jnp.int32(0))
            pltpu.async_copy(g, out_hbm.at[ci], ss)

        def wait_copy():
            @pl.when(wid < NFULL)
            def _():
                pltpu.make_async_copy(mem_hbm.at[pl.ds(lo, OWN)],
                                      out_hbm.at[pl.ds(lo, OWN)],
                                      sem_c).wait()

            @pl.when(wid == NFULL)
            def _():
                tlo = pl.multiple_of(NFULL * OWN, 8)
                pltpu.make_async_copy(mem_hbm.at[pl.ds(tlo, TAILN)],
                                      out_hbm.at[pl.ds(tlo, TAILN)],
                                      sem_c).wait()

        @pl.when(nwin > 0)
        def _process():
            head_i = sel_idx[pl.ds(nwin - 1, L)]
            head_p = sel_pos[pl.ds(nwin - 1, L)]
            fw_i = jnp.sum(jnp.where(lanes == 0, head_i, 0))
            fw_p = jnp.sum(jnp.where(lanes == 0, head_p, 0))
            for k in range(C // L):
                sel_idx[pl.ds(nwin + k * L, L)] = jnp.full((L,), fw_i,
                                                           jnp.int32)
                sel_pos[pl.ds(nwin + k * L, L)] = jnp.full((L,), fw_p,
                                                           jnp.int32)
            nch = pl.cdiv(nwin, C)

            stage(0, bufs[0])

            @pl.when(nch >= 2)
            def _():
                stage(1, bufs[1])

            wait_copy()

            def chunk_iter(c_, carry):
                for p in range(NBUF):
                    @pl.when(c_ % NBUF == p)
                    def _(p=p):
                        nxt = bufs[(p + 2) % NBUF]

                        @pl.when(c_ + 2 < nch)
                        def _():
                            @pl.when(c_ >= 1)
                            def _():
                                wait_scatter(nxt)
                            stage(c_ + 2, nxt)

                        blend_scatter(bufs[p])
                return carry

            lax.fori_loop(0, nch, chunk_iter, jnp.int32(0))

            for bi in range(NBUF):
                @pl.when(nch >= bi + 1)
                def _(bi=bi):
                    wait_scatter(bufs[bi])

        @pl.when(nwin == 0)
        def _nowork():
            wait_copy()

    return sc_update


def kernel(mem, idx, val):
    M, D = mem.shape
    (B,) = idx.shape
    sc_update = _build(M, D, B)
    return sc_update(mem, idx.astype(jnp.int32), val)


# speedup vs baseline: 18.8307x; 18.8307x over previous
import functools

import jax
import jax.numpy as jnp
from jax import lax
from jax.experimental import pallas as pl
from jax.experimental.pallas import tpu as pltpu
from jax.experimental.pallas import tpu_sc as plsc

MOMENTUM = 0.8
NC, NS, L = 2, 16, 16
NW = NC * NS
C = 64
NBUF = 3
SELU = 4


@functools.cache
def _build(M, D, B):
    OWN = pl.cdiv(M, NW)
    SEL = B + C + L
    mesh = plsc.VectorSubcoreMesh(
        core_axis_name="c", subcore_axis_name="s",
        num_cores=NC, num_subcores=NS)

    row_bufs = []
    for _ in range(NBUF):
        row_bufs += [
            pltpu.VMEM((C,), jnp.int32),
            pltpu.VMEM((C,), jnp.int32),
            pltpu.VMEM((C, D), jnp.float32),
            pltpu.VMEM((C, D), jnp.float32),
            pltpu.SemaphoreType.DMA,
            pltpu.SemaphoreType.DMA,
            pltpu.SemaphoreType.DMA,
        ]

    @functools.partial(
        pl.kernel, mesh=mesh, out_type=(),
        compiler_params=pltpu.CompilerParams(needs_layout_passes=False),
        scratch_types=[
            pltpu.VMEM((B,), jnp.int32),
            pltpu.VMEM((SEL,), jnp.int32),
            pltpu.VMEM((SEL,), jnp.int32),
            pltpu.VMEM((OWN + L,), jnp.int32),
        ] + row_bufs,
    )
    def sc_update(out_hbm, idx_hbm, val_hbm,
                  idx_v, sel_idx, sel_pos, aux, *bufflat):
        bufs = [tuple(bufflat[i * 7:(i + 1) * 7]) for i in range(NBUF)]
        wid = lax.axis_index("s") * NC + lax.axis_index("c")
        lo = wid * OWN
        hi = jnp.minimum(lo + OWN, M)
        lanes = lax.iota(jnp.int32, L)

        pltpu.sync_copy(idx_hbm, idx_v)

        def sel_body(i, off):
            for u in range(SELU):
                j = i * SELU + u
                v = idx_v[pl.ds(j * L, L)]
                msk = (v >= lo) & (v < hi)
                plsc.store_compressed(sel_idx.at[pl.ds(off, L)], v, mask=msk)
                plsc.store_compressed(sel_pos.at[pl.ds(off, L)],
                                      lanes + j * L, mask=msk)
                off = off + jnp.sum(msk.astype(jnp.int32))
            return off

        n = lax.fori_loop(0, B // (L * SELU), sel_body, jnp.int32(0))
        nvec = pl.cdiv(n, L)

        def _local(i):
            raw = sel_idx[pl.ds(i * L, L)]
            vl = jnp.minimum(jnp.maximum(raw - lo, 0), OWN - 1)
            vp = sel_pos[pl.ds(i * L, L)]
            valid = (lanes + i * L) < n
            return raw, vl, vp, valid

        def scat0(i, c_):
            _, vl, vp, valid = _local(i)
            plsc.store_scatter(aux, [vl], vp, mask=valid)
            return c_

        lax.fori_loop(0, nvec, scat0, jnp.int32(0))

        def fix_cond(state):
            return state[0] > 0

        def fix_body(state):
            _, it = state

            def chk(i, pend):
                _, vl, vp, valid = _local(i)
                cur = plsc.load_gather(aux, [vl])
                need = valid & (cur < vp)
                plsc.store_scatter(aux, [vl], vp, mask=need)
                return pend + jnp.sum(need.astype(jnp.int32))

            pend = lax.fori_loop(0, nvec, chk, jnp.int32(0))
            return pend, it + 1

        lax.while_loop(fix_cond, fix_body, (jnp.int32(1), jnp.int32(0)))

        def win_body(i, m):
            raw, vl, vp, valid = _local(i)
            cur = plsc.load_gather(aux, [vl])
            win = valid & (cur == vp)
            plsc.store_compressed(sel_idx.at[pl.ds(m, L)], raw, mask=win)
            plsc.store_compressed(sel_pos.at[pl.ds(m, L)], vp, mask=win)
            return m + jnp.sum(win.astype(jnp.int32))

        nwin = lax.fori_loop(0, nvec, win_body, jnp.int32(0))

        def stage(c_, buf):
            ci, cp, g, v, sg, sv, ss = buf
            for k in range(C // L):
                ci[pl.ds(k * L, L)] = sel_idx[pl.ds(c_ * C + k * L, L)]
                cp[pl.ds(k * L, L)] = sel_pos[pl.ds(c_ * C + k * L, L)]
            pltpu.async_copy(out_hbm.at[ci], g, sg)
            pltpu.async_copy(val_hbm.at[cp], v, sv)

        def wait_scatter(buf):
            ci, cp, g, v, sg, sv, ss = buf
            pltpu.make_async_copy(g, out_hbm.at[ci], ss).wait()

        def blend_scatter(buf):
            ci, cp, g, v, sg, sv, ss = buf
            pltpu.make_async_copy(out_hbm.at[ci], g, sg).wait()
            pltpu.make_async_copy(val_hbm.at[cp], v, sv).wait()

            def brow(r, carry):
                for d in range(D // L):
                    sl = pl.ds(d * L, L)
                    g[r, sl] = ((1.0 - MOMENTUM) * g[r, sl]
                                + MOMENTUM * v[r, sl])
                return carry

            lax.fori_loop(0, C, brow, jnp.int32(0))
            pltpu.async_copy(g, out_hbm.at[ci], ss)

        @pl.when(nwin > 0)
        def _process():
            head_i = sel_idx[pl.ds(nwin - 1, L)]
            head_p = sel_pos[pl.ds(nwin - 1, L)]
            fw_i = jnp.sum(jnp.where(lanes == 0, head_i, 0))
            fw_p = jnp.sum(jnp.where(lanes == 0, head_p, 0))
            for k in range(C // L):
                sel_idx[pl.ds(nwin + k * L, L)] = jnp.full((L,), fw_i,
                                                           jnp.int32)
                sel_pos[pl.ds(nwin + k * L, L)] = jnp.full((L,), fw_p,
                                                           jnp.int32)
            nch = pl.cdiv(nwin, C)

            stage(0, bufs[0])

            @pl.when(nch >= 2)
            def _():
                stage(1, bufs[1])

            def chunk_iter(c_, carry):
                for p in range(NBUF):
                    @pl.when(c_ % NBUF == p)
                    def _(p=p):
                        nxt = bufs[(p + 2) % NBUF]

                        @pl.when(c_ + 2 < nch)
                        def _():
                            @pl.when(c_ >= 1)
                            def _():
                                wait_scatter(nxt)
                            stage(c_ + 2, nxt)

                        blend_scatter(bufs[p])
                return carry

            lax.fori_loop(0, nch, chunk_iter, jnp.int32(0))

            for bi in range(NBUF):
                @pl.when(nch >= bi + 1)
                def _(bi=bi):
                    wait_scatter(bufs[bi])

    return sc_update


def kernel(mem, idx, val):
    M, D = mem.shape
    (B,) = idx.shape
    sc_update = _build(M, D, B)
    ref = jax.new_ref(mem)
    sc_update(ref, idx.astype(jnp.int32), val)
    return ref[...]
